# baseline (speedup 1.0000x reference)
import jax
import jax.numpy as jnp
from jax.experimental import pallas as pl
from jax.experimental.pallas import tpu as pltpu

D_MODEL = 768
D_STATE = 16
D_CONV = 4
D_INNER = 1536
DT_RANK = 48
L = 2048
LB = 256
PAD = 8


def _proj_kernel(x_ref, win_ref, cw_ref, cb_ref, xp_ref, dtw_ref, dtb_ref,
                 u_ref, gate_ref, dt_ref, b_ref, c_ref):
    i = pl.program_id(0)
    xb = x_ref[pl.ds(i * LB, LB + PAD), :]
    proj = jax.lax.dot_general(xb, win_ref[...], (((1,), (1,)), ((), ())),
                               preferred_element_type=jnp.float32)
    hidden = proj[:, :D_INNER]
    gate_ref[...] = proj[PAD:, D_INNER:]
    acc = cb_ref[...]
    for k in range(D_CONV):
        acc = acc + cw_ref[k:k + 1, :] * hidden[PAD - 3 + k:PAD - 3 + k + LB, :]
    u = acc * jax.nn.sigmoid(acc)
    u_ref[...] = u
    ssm = jax.lax.dot_general(u, xp_ref[...], (((1,), (1,)), ((), ())),
                              preferred_element_type=jnp.float32)
    dt_lin = jax.lax.dot_general(ssm[:, :DT_RANK], dtw_ref[...],
                                 (((1,), (1,)), ((), ())),
                                 preferred_element_type=jnp.float32)
    dt_ref[...] = jax.nn.softplus(dt_lin + dtb_ref[...])
    b_ref[...] = ssm[:, DT_RANK:DT_RANK + D_STATE]
    c_ref[...] = ssm[:, DT_RANK + D_STATE:DT_RANK + 2 * D_STATE]


TB = 64
DH = D_INNER // 2


def _outer(row16, row_d):
    return jax.lax.dot_general(row16, row_d, (((0,), (0,)), ((), ())),
                               preferred_element_type=jnp.float32)


def _contract(row16, h):
    return jax.lax.dot_general(row16, h, (((1,), (0,)), ((), ())),
                               preferred_element_type=jnp.float32)


def _scan_kernel(dt_ref, u_ref, b_ref, c_ref, alt_ref, yf_ref, yb_ref):
    Am = -jnp.exp(alt_ref[...])

    def _terms(i):
        dtr = dt_ref[pl.ds(i, 1), :]
        w = jnp.exp(Am * dtr)
        fi = _outer(b_ref[pl.ds(i, 1), :], dtr * u_ref[pl.ds(i, 1), :])
        return w, fi

    NC = L // TB

    def step(ci, hg):
        h, g = hg
        fb = pl.multiple_of(ci * TB, TB)
        bb = pl.multiple_of((NC - 1 - ci) * TB, TB)
        for t in range(TB):
            tb = TB - 1 - t
            wf, ff = _terms(fb + t)
            h = wf * h + ff
            yf_ref[pl.ds(fb + t, 1), :] = _contract(
                c_ref[pl.ds(fb + t, 1), :], h)
            yb_ref[pl.ds(bb + tb, 1), :] = _contract(
                c_ref[pl.ds(bb + tb, 1), :], g)
            wb, fbk = _terms(bb + tb)
            g = wb * (g + fbk)
        return (h, g)

    h0 = jnp.zeros((D_STATE, DH), jnp.float32)
    jax.lax.fori_loop(0, NC, step, (h0, h0))


def _out_kernel(yf_ref, yb_ref, u_ref, gate_ref, d_ref, wout_ref, o_ref):
    g = gate_ref[...]
    y = 1.3 * (yf_ref[...] + yb_ref[...]) + u_ref[...] * d_ref[...]
    s = y * (g * jax.nn.sigmoid(g))
    o_ref[...] = jax.lax.dot_general(s, wout_ref[...], (((1,), (1,)), ((), ())),
                                     preferred_element_type=jnp.float32)


def kernel(input_states, context_len, in_proj_w, conv_w, conv_b, x_proj_w,
           dt_proj_w, dt_proj_b, A_log, D, out_proj_w):
    del context_len
    x = input_states[0]
    x_pad = jnp.pad(x, ((PAD, 0), (0, 0)))
    cw = jnp.transpose(conv_w[:, 0, :], (1, 0))
    cb = conv_b[None, :]
    dtb = dt_proj_b[None, :]
    d_row = D[None, :]

    full = lambda shp: pl.BlockSpec(shp, lambda i: (0, 0))
    blk = lambda shp: pl.BlockSpec(shp, lambda i: (i, 0))

    u, gate, dt, Bm, Cm = pl.pallas_call(
        _proj_kernel,
        grid=(L // LB,),
        in_specs=[full((L + PAD, D_MODEL)), full((2 * D_INNER, D_MODEL)),
                  full((D_CONV, D_INNER)), full((1, D_INNER)),
                  full((DT_RANK + 2 * D_STATE, D_INNER)),
                  full((D_INNER, DT_RANK)), full((1, D_INNER))],
        out_specs=[blk((LB, D_INNER)), blk((LB, D_INNER)), blk((LB, D_INNER)),
                   blk((LB, D_STATE)), blk((LB, D_STATE))],
        out_shape=[jax.ShapeDtypeStruct((L, D_INNER), jnp.float32),
                   jax.ShapeDtypeStruct((L, D_INNER), jnp.float32),
                   jax.ShapeDtypeStruct((L, D_INNER), jnp.float32),
                   jax.ShapeDtypeStruct((L, D_STATE), jnp.float32),
                   jax.ShapeDtypeStruct((L, D_STATE), jnp.float32)],
        compiler_params=pltpu.CompilerParams(
            dimension_semantics=("parallel",)),
    )(x_pad, in_proj_w, cw, cb, x_proj_w, dt_proj_w, dtb)

    alt = jnp.transpose(A_log, (1, 0))

    dblk = lambda shp: pl.BlockSpec(shp, lambda i: (0, i))

    yf, yb = pl.pallas_call(
        _scan_kernel,
        grid=(D_INNER // DH,),
        in_specs=[dblk((L, DH)), dblk((L, DH)),
                  full((L, D_STATE)), full((L, D_STATE)),
                  dblk((D_STATE, DH))],
        out_specs=[dblk((L, DH)), dblk((L, DH))],
        out_shape=[jax.ShapeDtypeStruct((L, D_INNER), jnp.float32),
                   jax.ShapeDtypeStruct((L, D_INNER), jnp.float32)],
        compiler_params=pltpu.CompilerParams(
            dimension_semantics=("parallel",)),
    )(dt, u, Bm, Cm, alt)

    OB = 512
    out = pl.pallas_call(
        _out_kernel,
        grid=(L // OB,),
        in_specs=[pl.BlockSpec((OB, D_INNER), lambda i: (i, 0)),
                  pl.BlockSpec((OB, D_INNER), lambda i: (i, 0)),
                  pl.BlockSpec((OB, D_INNER), lambda i: (i, 0)),
                  pl.BlockSpec((OB, D_INNER), lambda i: (i, 0)),
                  pl.BlockSpec((1, D_INNER), lambda i: (0, 0)),
                  pl.BlockSpec((D_MODEL, D_INNER), lambda i: (0, 0))],
        out_specs=pl.BlockSpec((OB, D_MODEL), lambda i: (i, 0)),
        out_shape=jax.ShapeDtypeStruct((L, D_MODEL), jnp.float32),
        compiler_params=pltpu.CompilerParams(
            dimension_semantics=("parallel",)),
    )(yf, yb, u, gate, d_row, out_proj_w)

    return out[None]


# speedup vs baseline: 2.7853x; 2.7853x over previous
import jax
import jax.numpy as jnp
from jax.experimental import pallas as pl
from jax.experimental.pallas import tpu as pltpu

D_MODEL = 768
D_STATE = 16
D_CONV = 4
D_INNER = 1536
DT_RANK = 48
L = 2048
LB = 256
PAD = 8


def _proj_kernel(x_ref, win_ref, cw_ref, cb_ref, xp_ref, dtw_ref, dtb_ref,
                 u_ref, gate_ref, dt_ref, b_ref, c_ref):
    i = pl.program_id(0)
    xb = x_ref[pl.ds(i * LB, LB + PAD), :]
    proj = jax.lax.dot_general(xb, win_ref[...], (((1,), (1,)), ((), ())),
                               preferred_element_type=jnp.float32)
    hidden = proj[:, :D_INNER]
    gate_ref[...] = proj[PAD:, D_INNER:]
    acc = cb_ref[...]
    for k in range(D_CONV):
        acc = acc + cw_ref[k:k + 1, :] * hidden[PAD - 3 + k:PAD - 3 + k + LB, :]
    u = acc * jax.nn.sigmoid(acc)
    u_ref[...] = u
    ssm = jax.lax.dot_general(u, xp_ref[...], (((1,), (1,)), ((), ())),
                              preferred_element_type=jnp.float32)
    dt_lin = jax.lax.dot_general(ssm[:, :DT_RANK], dtw_ref[...],
                                 (((1,), (1,)), ((), ())),
                                 preferred_element_type=jnp.float32)
    dt_ref[...] = jax.nn.softplus(dt_lin + dtb_ref[...])
    b_ref[...] = ssm[:, DT_RANK:DT_RANK + D_STATE]
    c_ref[...] = ssm[:, DT_RANK + D_STATE:DT_RANK + 2 * D_STATE]


TB = 64
SB = 8
DH = D_INNER // 2


def _outer(row16, row_d):
    return jax.lax.dot_general(row16, row_d, (((0,), (0,)), ((), ())),
                               preferred_element_type=jnp.float32)


def _contract(row16, h):
    return jax.lax.dot_general(row16, h, (((1,), (0,)), ((), ())),
                               preferred_element_type=jnp.float32)


def _scan_kernel(dt_ref, u_ref, b_ref, c_ref, alt_ref, yf_ref, yb_ref):
    Am = -jnp.exp(alt_ref[...])

    def _sub_terms(base):
        dt_c = dt_ref[pl.ds(base, SB), :]
        dtu = dt_c * u_ref[pl.ds(base, SB), :]
        b_c = b_ref[pl.ds(base, SB), :]
        c_c = c_ref[pl.ds(base, SB), :]
        ws = [jnp.exp(Am * dt_c[t:t + 1, :]) for t in range(SB)]
        fis = [_outer(b_c[t:t + 1, :], dtu[t:t + 1, :]) for t in range(SB)]
        return c_c, ws, fis

    NC = L // TB

    def step(ci, hg):
        h, g = hg
        fb = pl.multiple_of(ci * TB, TB)
        bb = pl.multiple_of((NC - 1 - ci) * TB, TB)
        for sb in range(TB // SB):
            f0 = fb + sb * SB
            b0 = bb + TB - (sb + 1) * SB
            c_f, ws_f, fis_f = _sub_terms(f0)
            c_b, ws_b, fis_b = _sub_terms(b0)
            for t in range(SB):
                tb = SB - 1 - t
                h = ws_f[t] * h + fis_f[t]
                yf_ref[pl.ds(f0 + t, 1), :] = _contract(c_f[t:t + 1, :], h)
                yb_ref[pl.ds(b0 + tb, 1), :] = _contract(c_b[tb:tb + 1, :], g)
                g = ws_b[tb] * (g + fis_b[tb])
        return (h, g)

    h0 = jnp.zeros((D_STATE, DH), jnp.float32)
    jax.lax.fori_loop(0, NC, step, (h0, h0))


def _out_kernel(yf_ref, yb_ref, u_ref, gate_ref, d_ref, wout_ref, o_ref):
    g = gate_ref[...]
    y = 1.3 * (yf_ref[...] + yb_ref[...]) + u_ref[...] * d_ref[...]
    s = y * (g * jax.nn.sigmoid(g))
    o_ref[...] = jax.lax.dot_general(s, wout_ref[...], (((1,), (1,)), ((), ())),
                                     preferred_element_type=jnp.float32)


def kernel(input_states, context_len, in_proj_w, conv_w, conv_b, x_proj_w,
           dt_proj_w, dt_proj_b, A_log, D, out_proj_w):
    del context_len
    x = input_states[0]
    x_pad = jnp.pad(x, ((PAD, 0), (0, 0)))
    cw = jnp.transpose(conv_w[:, 0, :], (1, 0))
    cb = conv_b[None, :]
    dtb = dt_proj_b[None, :]
    d_row = D[None, :]

    full = lambda shp: pl.BlockSpec(shp, lambda i: (0, 0))
    blk = lambda shp: pl.BlockSpec(shp, lambda i: (i, 0))

    u, gate, dt, Bm, Cm = pl.pallas_call(
        _proj_kernel,
        grid=(L // LB,),
        in_specs=[full((L + PAD, D_MODEL)), full((2 * D_INNER, D_MODEL)),
                  full((D_CONV, D_INNER)), full((1, D_INNER)),
                  full((DT_RANK + 2 * D_STATE, D_INNER)),
                  full((D_INNER, DT_RANK)), full((1, D_INNER))],
        out_specs=[blk((LB, D_INNER)), blk((LB, D_INNER)), blk((LB, D_INNER)),
                   blk((LB, D_STATE)), blk((LB, D_STATE))],
        out_shape=[jax.ShapeDtypeStruct((L, D_INNER), jnp.float32),
                   jax.ShapeDtypeStruct((L, D_INNER), jnp.float32),
                   jax.ShapeDtypeStruct((L, D_INNER), jnp.float32),
                   jax.ShapeDtypeStruct((L, D_STATE), jnp.float32),
                   jax.ShapeDtypeStruct((L, D_STATE), jnp.float32)],
        compiler_params=pltpu.CompilerParams(
            dimension_semantics=("parallel",)),
    )(x_pad, in_proj_w, cw, cb, x_proj_w, dt_proj_w, dtb)

    alt = jnp.transpose(A_log, (1, 0))

    dblk = lambda shp: pl.BlockSpec(shp, lambda i: (0, i))

    yf, yb = pl.pallas_call(
        _scan_kernel,
        grid=(D_INNER // DH,),
        in_specs=[dblk((L, DH)), dblk((L, DH)),
                  full((L, D_STATE)), full((L, D_STATE)),
                  dblk((D_STATE, DH))],
        out_specs=[dblk((L, DH)), dblk((L, DH))],
        out_shape=[jax.ShapeDtypeStruct((L, D_INNER), jnp.float32),
                   jax.ShapeDtypeStruct((L, D_INNER), jnp.float32)],
        compiler_params=pltpu.CompilerParams(
            dimension_semantics=("parallel",)),
    )(dt, u, Bm, Cm, alt)

    OB = 512
    out = pl.pallas_call(
        _out_kernel,
        grid=(L // OB,),
        in_specs=[pl.BlockSpec((OB, D_INNER), lambda i: (i, 0)),
                  pl.BlockSpec((OB, D_INNER), lambda i: (i, 0)),
                  pl.BlockSpec((OB, D_INNER), lambda i: (i, 0)),
                  pl.BlockSpec((OB, D_INNER), lambda i: (i, 0)),
                  pl.BlockSpec((1, D_INNER), lambda i: (0, 0)),
                  pl.BlockSpec((D_MODEL, D_INNER), lambda i: (0, 0))],
        out_specs=pl.BlockSpec((OB, D_MODEL), lambda i: (i, 0)),
        out_shape=jax.ShapeDtypeStruct((L, D_MODEL), jnp.float32),
        compiler_params=pltpu.CompilerParams(
            dimension_semantics=("parallel",)),
    )(yf, yb, u, gate, d_row, out_proj_w)

    return out[None]


# speedup vs baseline: 3.1198x; 1.1201x over previous
import jax
import jax.numpy as jnp
from jax.experimental import pallas as pl
from jax.experimental.pallas import tpu as pltpu

D_MODEL = 768
D_STATE = 16
D_CONV = 4
D_INNER = 1536
DT_RANK = 48
L = 2048
LB = 256
PAD = 8


def _proj_kernel(x_ref, win_ref, cw_ref, cb_ref, xp_ref, dtw_ref, dtb_ref,
                 u_ref, gate_ref, dt_ref, b_ref, c_ref):
    i = pl.program_id(0)
    xb = x_ref[pl.ds(i * LB, LB + PAD), :]
    proj = jax.lax.dot_general(xb, win_ref[...], (((1,), (1,)), ((), ())),
                               preferred_element_type=jnp.float32)
    hidden = proj[:, :D_INNER]
    gate_ref[...] = proj[PAD:, D_INNER:]
    acc = cb_ref[...]
    for k in range(D_CONV):
        acc = acc + cw_ref[k:k + 1, :] * hidden[PAD - 3 + k:PAD - 3 + k + LB, :]
    u = acc * jax.nn.sigmoid(acc)
    u_ref[...] = u
    ssm = jax.lax.dot_general(u, xp_ref[...], (((1,), (1,)), ((), ())),
                              preferred_element_type=jnp.float32)
    dt_lin = jax.lax.dot_general(ssm[:, :DT_RANK], dtw_ref[...],
                                 (((1,), (1,)), ((), ())),
                                 preferred_element_type=jnp.float32)
    dt_ref[...] = jax.nn.softplus(dt_lin + dtb_ref[...])
    b_ref[...] = ssm[:, DT_RANK:DT_RANK + D_STATE]
    c_ref[...] = ssm[:, DT_RANK + D_STATE:DT_RANK + 2 * D_STATE]


TB = 64
SB = 16
DH = D_INNER // 2


def _outer(row16, row_d):
    return jax.lax.dot_general(row16, row_d, (((0,), (0,)), ((), ())),
                               preferred_element_type=jnp.float32)


def _contract(row16, h):
    return jax.lax.dot_general(row16, h, (((1,), (0,)), ((), ())),
                               preferred_element_type=jnp.float32)


def _scan_kernel(dt_ref, u_ref, b_ref, c_ref, alt_ref, yf_ref, yb_ref):
    Am = -jnp.exp(alt_ref[...])

    def _sub_terms(base):
        dt_c = dt_ref[pl.ds(base, SB), :]
        dtu = dt_c * u_ref[pl.ds(base, SB), :]
        b_c = b_ref[pl.ds(base, SB), :]
        c_c = c_ref[pl.ds(base, SB), :]
        ws = [jnp.exp(Am * dt_c[t:t + 1, :]) for t in range(SB)]
        fis = [_outer(b_c[t:t + 1, :], dtu[t:t + 1, :]) for t in range(SB)]
        return c_c, ws, fis

    NC = L // TB

    def step(ci, hg):
        h, g = hg
        fb = pl.multiple_of(ci * TB, TB)
        bb = pl.multiple_of((NC - 1 - ci) * TB, TB)
        for sb in range(TB // SB):
            f0 = fb + sb * SB
            b0 = bb + TB - (sb + 1) * SB
            c_f, ws_f, fis_f = _sub_terms(f0)
            c_b, ws_b, fis_b = _sub_terms(b0)
            for t in range(SB):
                tb = SB - 1 - t
                h = ws_f[t] * h + fis_f[t]
                yf_ref[pl.ds(f0 + t, 1), :] = _contract(c_f[t:t + 1, :], h)
                yb_ref[pl.ds(b0 + tb, 1), :] = _contract(c_b[tb:tb + 1, :], g)
                g = ws_b[tb] * (g + fis_b[tb])
        return (h, g)

    h0 = jnp.zeros((D_STATE, DH), jnp.float32)
    jax.lax.fori_loop(0, NC, step, (h0, h0))


def _out_kernel(yf_ref, yb_ref, u_ref, gate_ref, d_ref, wout_ref, o_ref):
    g = gate_ref[...]
    y = 1.3 * (yf_ref[...] + yb_ref[...]) + u_ref[...] * d_ref[...]
    s = y * (g * jax.nn.sigmoid(g))
    o_ref[...] = jax.lax.dot_general(s, wout_ref[...], (((1,), (1,)), ((), ())),
                                     preferred_element_type=jnp.float32)


def kernel(input_states, context_len, in_proj_w, conv_w, conv_b, x_proj_w,
           dt_proj_w, dt_proj_b, A_log, D, out_proj_w):
    del context_len
    x = input_states[0]
    x_pad = jnp.pad(x, ((PAD, 0), (0, 0)))
    cw = jnp.transpose(conv_w[:, 0, :], (1, 0))
    cb = conv_b[None, :]
    dtb = dt_proj_b[None, :]
    d_row = D[None, :]

    full = lambda shp: pl.BlockSpec(shp, lambda i: (0, 0))
    blk = lambda shp: pl.BlockSpec(shp, lambda i: (i, 0))

    u, gate, dt, Bm, Cm = pl.pallas_call(
        _proj_kernel,
        grid=(L // LB,),
        in_specs=[full((L + PAD, D_MODEL)), full((2 * D_INNER, D_MODEL)),
                  full((D_CONV, D_INNER)), full((1, D_INNER)),
                  full((DT_RANK + 2 * D_STATE, D_INNER)),
                  full((D_INNER, DT_RANK)), full((1, D_INNER))],
        out_specs=[blk((LB, D_INNER)), blk((LB, D_INNER)), blk((LB, D_INNER)),
                   blk((LB, D_STATE)), blk((LB, D_STATE))],
        out_shape=[jax.ShapeDtypeStruct((L, D_INNER), jnp.float32),
                   jax.ShapeDtypeStruct((L, D_INNER), jnp.float32),
                   jax.ShapeDtypeStruct((L, D_INNER), jnp.float32),
                   jax.ShapeDtypeStruct((L, D_STATE), jnp.float32),
                   jax.ShapeDtypeStruct((L, D_STATE), jnp.float32)],
        compiler_params=pltpu.CompilerParams(
            dimension_semantics=("parallel",)),
    )(x_pad, in_proj_w, cw, cb, x_proj_w, dt_proj_w, dtb)

    alt = jnp.transpose(A_log, (1, 0))

    dblk = lambda shp: pl.BlockSpec(shp, lambda i: (0, i))

    yf, yb = pl.pallas_call(
        _scan_kernel,
        grid=(D_INNER // DH,),
        in_specs=[dblk((L, DH)), dblk((L, DH)),
                  full((L, D_STATE)), full((L, D_STATE)),
                  dblk((D_STATE, DH))],
        out_specs=[dblk((L, DH)), dblk((L, DH))],
        out_shape=[jax.ShapeDtypeStruct((L, D_INNER), jnp.float32),
                   jax.ShapeDtypeStruct((L, D_INNER), jnp.float32)],
        compiler_params=pltpu.CompilerParams(
            dimension_semantics=("parallel",)),
    )(dt, u, Bm, Cm, alt)

    OB = 512
    out = pl.pallas_call(
        _out_kernel,
        grid=(L // OB,),
        in_specs=[pl.BlockSpec((OB, D_INNER), lambda i: (i, 0)),
                  pl.BlockSpec((OB, D_INNER), lambda i: (i, 0)),
                  pl.BlockSpec((OB, D_INNER), lambda i: (i, 0)),
                  pl.BlockSpec((OB, D_INNER), lambda i: (i, 0)),
                  pl.BlockSpec((1, D_INNER), lambda i: (0, 0)),
                  pl.BlockSpec((D_MODEL, D_INNER), lambda i: (0, 0))],
        out_specs=pl.BlockSpec((OB, D_MODEL), lambda i: (i, 0)),
        out_shape=jax.ShapeDtypeStruct((L, D_MODEL), jnp.float32),
        compiler_params=pltpu.CompilerParams(
            dimension_semantics=("parallel",)),
    )(yf, yb, u, gate, d_row, out_proj_w)

    return out[None]


# speedup vs baseline: 3.1640x; 1.0142x over previous
import jax
import jax.numpy as jnp
from jax.experimental import pallas as pl
from jax.experimental.pallas import tpu as pltpu

D_MODEL = 768
D_STATE = 16
D_CONV = 4
D_INNER = 1536
DT_RANK = 48
L = 2048
LB = 256
PAD = 8


def _proj_kernel(x_ref, win_ref, cw_ref, cb_ref, xp_ref, dtw_ref, dtb_ref,
                 u_ref, gate_ref, dt_ref, b_ref, c_ref):
    i = pl.program_id(0)
    xb = x_ref[pl.ds(i * LB, LB + PAD), :]
    proj = jax.lax.dot_general(xb, win_ref[...], (((1,), (1,)), ((), ())),
                               preferred_element_type=jnp.float32)
    hidden = proj[:, :D_INNER]
    gate_ref[...] = proj[PAD:, D_INNER:]
    acc = cb_ref[...]
    for k in range(D_CONV):
        acc = acc + cw_ref[k:k + 1, :] * hidden[PAD - 3 + k:PAD - 3 + k + LB, :]
    u = acc * jax.nn.sigmoid(acc)
    u_ref[...] = u
    ssm = jax.lax.dot_general(u, xp_ref[...], (((1,), (1,)), ((), ())),
                              preferred_element_type=jnp.float32)
    dt_lin = jax.lax.dot_general(ssm[:, :DT_RANK], dtw_ref[...],
                                 (((1,), (1,)), ((), ())),
                                 preferred_element_type=jnp.float32)
    dt_ref[...] = jax.nn.softplus(dt_lin + dtb_ref[...])
    b_ref[...] = ssm[:, DT_RANK:DT_RANK + D_STATE]
    c_ref[...] = ssm[:, DT_RANK + D_STATE:DT_RANK + 2 * D_STATE]


TB = 64
SB = 64
DH = D_INNER // 2


def _outer(row16, row_d):
    return jax.lax.dot_general(row16, row_d, (((0,), (0,)), ((), ())),
                               preferred_element_type=jnp.float32)


def _contract(row16, h):
    return jax.lax.dot_general(row16, h, (((1,), (0,)), ((), ())),
                               preferred_element_type=jnp.float32)


def _scan_kernel(dt_ref, u_ref, b_ref, c_ref, alt_ref, yf_ref, yb_ref):
    Am = -jnp.exp(alt_ref[...])

    def _sub_terms(base):
        dt_c = dt_ref[pl.ds(base, SB), :]
        dtu = dt_c * u_ref[pl.ds(base, SB), :]
        b_c = b_ref[pl.ds(base, SB), :]
        c_c = c_ref[pl.ds(base, SB), :]
        ws = [jnp.exp(Am * dt_c[t:t + 1, :]) for t in range(SB)]
        fis = [_outer(b_c[t:t + 1, :], dtu[t:t + 1, :]) for t in range(SB)]
        return c_c, ws, fis

    NC = L // TB

    def step(ci, hg):
        h, g = hg
        fb = pl.multiple_of(ci * TB, TB)
        bb = pl.multiple_of((NC - 1 - ci) * TB, TB)
        for sb in range(TB // SB):
            f0 = fb + sb * SB
            b0 = bb + TB - (sb + 1) * SB
            c_f, ws_f, fis_f = _sub_terms(f0)
            c_b, ws_b, fis_b = _sub_terms(b0)
            for t in range(SB):
                tb = SB - 1 - t
                h = ws_f[t] * h + fis_f[t]
                yf_ref[pl.ds(f0 + t, 1), :] = _contract(c_f[t:t + 1, :], h)
                yb_ref[pl.ds(b0 + tb, 1), :] = _contract(c_b[tb:tb + 1, :], g)
                g = ws_b[tb] * (g + fis_b[tb])
        return (h, g)

    h0 = jnp.zeros((D_STATE, DH), jnp.float32)
    jax.lax.fori_loop(0, NC, step, (h0, h0))


def _out_kernel(yf_ref, yb_ref, u_ref, gate_ref, d_ref, wout_ref, o_ref):
    g = gate_ref[...]
    y = 1.3 * (yf_ref[...] + yb_ref[...]) + u_ref[...] * d_ref[...]
    s = y * (g * jax.nn.sigmoid(g))
    o_ref[...] = jax.lax.dot_general(s, wout_ref[...], (((1,), (1,)), ((), ())),
                                     preferred_element_type=jnp.float32)


def kernel(input_states, context_len, in_proj_w, conv_w, conv_b, x_proj_w,
           dt_proj_w, dt_proj_b, A_log, D, out_proj_w):
    del context_len
    x = input_states[0]
    x_pad = jnp.pad(x, ((PAD, 0), (0, 0)))
    cw = jnp.transpose(conv_w[:, 0, :], (1, 0))
    cb = conv_b[None, :]
    dtb = dt_proj_b[None, :]
    d_row = D[None, :]

    full = lambda shp: pl.BlockSpec(shp, lambda i: (0, 0))
    blk = lambda shp: pl.BlockSpec(shp, lambda i: (i, 0))

    u, gate, dt, Bm, Cm = pl.pallas_call(
        _proj_kernel,
        grid=(L // LB,),
        in_specs=[full((L + PAD, D_MODEL)), full((2 * D_INNER, D_MODEL)),
                  full((D_CONV, D_INNER)), full((1, D_INNER)),
                  full((DT_RANK + 2 * D_STATE, D_INNER)),
                  full((D_INNER, DT_RANK)), full((1, D_INNER))],
        out_specs=[blk((LB, D_INNER)), blk((LB, D_INNER)), blk((LB, D_INNER)),
                   blk((LB, D_STATE)), blk((LB, D_STATE))],
        out_shape=[jax.ShapeDtypeStruct((L, D_INNER), jnp.float32),
                   jax.ShapeDtypeStruct((L, D_INNER), jnp.float32),
                   jax.ShapeDtypeStruct((L, D_INNER), jnp.float32),
                   jax.ShapeDtypeStruct((L, D_STATE), jnp.float32),
                   jax.ShapeDtypeStruct((L, D_STATE), jnp.float32)],
        compiler_params=pltpu.CompilerParams(
            dimension_semantics=("parallel",)),
    )(x_pad, in_proj_w, cw, cb, x_proj_w, dt_proj_w, dtb)

    alt = jnp.transpose(A_log, (1, 0))

    dblk = lambda shp: pl.BlockSpec(shp, lambda i: (0, i))

    yf, yb = pl.pallas_call(
        _scan_kernel,
        grid=(D_INNER // DH,),
        in_specs=[dblk((L, DH)), dblk((L, DH)),
                  full((L, D_STATE)), full((L, D_STATE)),
                  dblk((D_STATE, DH))],
        out_specs=[dblk((L, DH)), dblk((L, DH))],
        out_shape=[jax.ShapeDtypeStruct((L, D_INNER), jnp.float32),
                   jax.ShapeDtypeStruct((L, D_INNER), jnp.float32)],
        compiler_params=pltpu.CompilerParams(
            dimension_semantics=("parallel",)),
    )(dt, u, Bm, Cm, alt)

    OB = 512
    out = pl.pallas_call(
        _out_kernel,
        grid=(L // OB,),
        in_specs=[pl.BlockSpec((OB, D_INNER), lambda i: (i, 0)),
                  pl.BlockSpec((OB, D_INNER), lambda i: (i, 0)),
                  pl.BlockSpec((OB, D_INNER), lambda i: (i, 0)),
                  pl.BlockSpec((OB, D_INNER), lambda i: (i, 0)),
                  pl.BlockSpec((1, D_INNER), lambda i: (0, 0)),
                  pl.BlockSpec((D_MODEL, D_INNER), lambda i: (0, 0))],
        out_specs=pl.BlockSpec((OB, D_MODEL), lambda i: (i, 0)),
        out_shape=jax.ShapeDtypeStruct((L, D_MODEL), jnp.float32),
        compiler_params=pltpu.CompilerParams(
            dimension_semantics=("parallel",)),
    )(yf, yb, u, gate, d_row, out_proj_w)

    return out[None]
